# baseline (speedup 1.0000x reference)
import functools

import jax
import jax.numpy as jnp
from jax import lax
from jax.experimental import pallas as pl
from jax.experimental.pallas import tpu as pltpu
from jax.experimental.pallas import tpu_sc as plsc

SCALE = 32.0


@functools.lru_cache(maxsize=None)
def _make_kernel(BATCH: int, S: int, D: int):
    info = plsc.get_sparse_core_info()
    NC, NS, L = info.num_cores, info.num_subcores, info.num_lanes
    NW = NC * NS
    B = BATCH * S
    assert B % NW == 0
    b_per_w = B // NW
    assert S % b_per_w == 0
    runs_per_row = S // b_per_w
    C = 8
    NBUF = 8
    assert b_per_w % (C * NBUF) == 0
    n_chunks = b_per_w // C
    n_outer = n_chunks // NBUF
    mesh = plsc.VectorSubcoreMesh(core_axis_name="c", subcore_axis_name="s")

    @functools.partial(
        pl.kernel,
        out_type=jax.ShapeDtypeStruct((BATCH, S, D), jnp.float32),
        mesh=mesh,
        scratch_types=[
            pltpu.VMEM((b_per_w,), jnp.int32),
            pltpu.VMEM((NBUF, C, D), jnp.float32),
        ] + [pltpu.SemaphoreType.DMA] * 16,
    )
    def k(x_hbm, table_hbm, out_hbm, idx_v, rows_v, *sems):
        gsem = sems[:NBUF]
        wsem = sems[NBUF:]
        wid = lax.axis_index("s") * NC + lax.axis_index("c")
        row = wid // runs_per_row
        col = (wid % runs_per_row) * b_per_w
        pltpu.sync_copy(x_hbm.at[row, pl.ds(col, b_per_w)], idx_v)

        def start_gather(c, b):
            pltpu.async_copy(
                table_hbm.at[idx_v.at[pl.ds(c * C, C)]], rows_v.at[b], gsem[b]
            )

        def wait_gather(b):
            pltpu.make_async_copy(
                table_hbm.at[idx_v.at[pl.ds(0, C)]], rows_v.at[b], gsem[b]
            ).wait()

        def start_write(c, b):
            pltpu.async_copy(
                rows_v.at[b], out_hbm.at[row, pl.ds(col + c * C, C)], wsem[b]
            )

        def wait_write(b):
            pltpu.make_async_copy(
                rows_v.at[b], out_hbm.at[0, pl.ds(0, C)], wsem[b]
            ).wait()

        start_gather(0, 0)
        start_gather(1, 1)

        def outer(g, carry):
            for b in range(NBUF):
                c = g * NBUF + b
                nb = (b + 2) % NBUF
                not_last = True if b < NBUF - 2 else g < n_outer - 1
                needs_drain = g >= 1 if b < NBUF - 2 else True

                def prefetch(nb=nb, c=c, needs_drain=needs_drain):
                    if needs_drain is True:
                        wait_write(nb)
                    else:

                        @pl.when(needs_drain)
                        def _():
                            wait_write(nb)

                    start_gather(c + 2, nb)

                if not_last is True:
                    prefetch()
                else:
                    pl.when(not_last)(prefetch)

                wait_gather(b)

                @plsc.parallel_loop(0, C, step=1, unroll=1)
                def _scale(r, b=b):
                    for j in range(D // L):
                        rows_v[b, r, pl.ds(j * L, L)] = (
                            rows_v[b, r, pl.ds(j * L, L)] * SCALE
                        )

                start_write(c, b)
            return carry

        lax.fori_loop(0, n_outer, outer, 0)
        for b in range(NBUF):
            wait_write(b)

    return k


@jax.jit
def kernel(x, table):
    batch, s = x.shape
    return _make_kernel(batch, s, table.shape[1])(x, table)


# speedup vs baseline: 1.0392x; 1.0392x over previous
import functools

import jax
import jax.numpy as jnp
from jax import lax
from jax.experimental import pallas as pl
from jax.experimental.pallas import tpu as pltpu
from jax.experimental.pallas import tpu_sc as plsc

SCALE = 32.0


@functools.lru_cache(maxsize=None)
def _make_kernel(BATCH: int, S: int, D: int):
    info = plsc.get_sparse_core_info()
    NC, NS, L = info.num_cores, info.num_subcores, info.num_lanes
    NW = NC * NS
    B = BATCH * S
    assert B % NW == 0
    b_per_w = B // NW
    assert S % b_per_w == 0
    runs_per_row = S // b_per_w
    C = 16
    NBUF = 4
    assert b_per_w % (C * NBUF) == 0
    n_chunks = b_per_w // C
    n_outer = n_chunks // NBUF
    mesh = plsc.VectorSubcoreMesh(core_axis_name="c", subcore_axis_name="s")

    @functools.partial(
        pl.kernel,
        out_type=jax.ShapeDtypeStruct((BATCH, S, D), jnp.float32),
        mesh=mesh,
        scratch_types=[
            pltpu.VMEM((b_per_w,), jnp.int32),
            pltpu.VMEM((NBUF, C, D), jnp.float32),
        ] + [pltpu.SemaphoreType.DMA] * 8,
    )
    def k(x_hbm, table_hbm, out_hbm, idx_v, rows_v, *sems):
        gsem = sems[:NBUF]
        wsem = sems[NBUF:]
        wid = lax.axis_index("s") * NC + lax.axis_index("c")
        row = wid // runs_per_row
        col = (wid % runs_per_row) * b_per_w
        pltpu.sync_copy(x_hbm.at[row, pl.ds(col, b_per_w)], idx_v)

        def start_gather(c, b):
            pltpu.async_copy(
                table_hbm.at[idx_v.at[pl.ds(c * C, C)]], rows_v.at[b], gsem[b]
            )

        def wait_gather(b):
            pltpu.make_async_copy(
                table_hbm.at[idx_v.at[pl.ds(0, C)]], rows_v.at[b], gsem[b]
            ).wait()

        def start_write(c, b):
            pltpu.async_copy(
                rows_v.at[b], out_hbm.at[row, pl.ds(col + c * C, C)], wsem[b]
            )

        def wait_write(b):
            pltpu.make_async_copy(
                rows_v.at[b], out_hbm.at[0, pl.ds(0, C)], wsem[b]
            ).wait()

        start_gather(0, 0)
        start_gather(1, 1)

        def outer(g, carry):
            for b in range(NBUF):
                c = g * NBUF + b
                nb = (b + 2) % NBUF
                not_last = True if b < NBUF - 2 else g < n_outer - 1
                needs_drain = g >= 1 if b < NBUF - 2 else True

                def prefetch(nb=nb, c=c, needs_drain=needs_drain):
                    if needs_drain is True:
                        wait_write(nb)
                    else:

                        @pl.when(needs_drain)
                        def _():
                            wait_write(nb)

                    start_gather(c + 2, nb)

                if not_last is True:
                    prefetch()
                else:
                    pl.when(not_last)(prefetch)

                wait_gather(b)

                @plsc.parallel_loop(0, C, step=1, unroll=1)
                def _scale(r, b=b):
                    for j in range(D // L):
                        rows_v[b, r, pl.ds(j * L, L)] = (
                            rows_v[b, r, pl.ds(j * L, L)] * SCALE
                        )

                start_write(c, b)
            return carry

        lax.fori_loop(0, n_outer, outer, 0)
        for b in range(NBUF):
            wait_write(b)

    return k


@jax.jit
def kernel(x, table):
    batch, s = x.shape
    return _make_kernel(batch, s, table.shape[1])(x, table)
